# Initial kernel scaffold; baseline (speedup 1.0000x reference)
import jax
import jax.numpy as jnp
from jax.experimental import pallas as pl


def kernel(x, base_W, base_b, router_W, router_b, lora_A, lora_B):
    raise NotImplementedError("write your pallas kernel here")


# speedup vs baseline: 7.6958x; 7.6958x over previous
import functools

import jax
import jax.numpy as jnp
from jax import lax
from jax.experimental import pallas as pl
from jax.experimental.pallas import tpu as pltpu
from jax.experimental.pallas import tpu_sc as plsc

_SCALING = 16.0 / 8.0


def _router_body(x_ref, rw_ref, rb_ref, lgT_ref):
    xf = x_ref[...]
    lgT = lax.dot_general(rw_ref[...], xf, (((1,), (1,)), ((), ())),
                          preferred_element_type=jnp.float32)
    lgT_ref[...] = lgT + rb_ref[...]


def _router_logits_T(xf, router_W, rb2, N, D, E, TB):
    grid = N // TB
    return pl.pallas_call(
        _router_body,
        grid=(grid,),
        in_specs=[
            pl.BlockSpec((TB, D), lambda i: (i, 0)),
            pl.BlockSpec((E, D), lambda i: (0, 0)),
            pl.BlockSpec((E, 1), lambda i: (0, 0)),
        ],
        out_specs=pl.BlockSpec((E, TB), lambda i: (0, i)),
        out_shape=jax.ShapeDtypeStruct((E, N), jnp.float32),
    )(xf, router_W, rb2)


def _make_sc_route(N, E):
    info = plsc.get_sparse_core_info()
    NC, NS, L = info.num_cores, info.num_subcores, info.num_lanes
    NW = NC * NS
    CW = N // NW
    mesh = plsc.VectorSubcoreMesh(core_axis_name="c", subcore_axis_name="s")

    @functools.partial(
        pl.kernel, mesh=mesh,
        out_type=jax.ShapeDtypeStruct((E, N), jnp.float32),
        scratch_types=[
            pltpu.VMEM((E, CW), jnp.float32),
            pltpu.VMEM((E, CW), jnp.float32),
        ],
    )
    def route(lgT_hbm, wT_hbm, lg_v, w_v):
        wid = lax.axis_index("s") * NC + lax.axis_index("c")
        base = wid * CW
        pltpu.sync_copy(lgT_hbm.at[:, pl.ds(base, CW)], lg_v)

        def chunk(c, carry):
            off = c * L
            Lg = [lg_v[e, pl.ds(off, L)] for e in range(E)]
            m1 = Lg[0]
            for e in range(1, E):
                m1 = jnp.maximum(m1, Lg[e])
            i1 = jnp.where(Lg[0] == m1, jnp.float32(0.0), jnp.float32(E))
            for e in range(1, E):
                i1 = jnp.minimum(i1, jnp.where(Lg[e] == m1, jnp.float32(e),
                                               jnp.float32(E)))
            L2 = [jnp.where(i1 == e, jnp.float32(-1e30), Lg[e])
                  for e in range(E)]
            m2 = L2[0]
            for e in range(1, E):
                m2 = jnp.maximum(m2, L2[e])
            i2 = jnp.where(L2[0] == m2, jnp.float32(0.0), jnp.float32(E))
            for e in range(1, E):
                i2 = jnp.minimum(i2, jnp.where(L2[e] == m2, jnp.float32(e),
                                               jnp.float32(E)))
            g = jnp.exp(m2 - m1)
            w1 = 1.0 / (1.0 + g)
            w2 = 1.0 - w1
            for e in range(E):
                w_v[e, pl.ds(off, L)] = jnp.where(
                    i1 == e, w1, jnp.where(i2 == e, w2, jnp.float32(0.0)))
            return carry

        lax.fori_loop(0, CW // L, chunk, jnp.int32(0))
        pltpu.sync_copy(w_v, wT_hbm.at[:, pl.ds(base, CW)])

    return route


def _main_body(x_ref, W_ref, b_ref, A_ref, Bf_ref, wT_ref, Ex_ref, o_ref):
    xb = x_ref[...].astype(jnp.bfloat16)
    acc = lax.dot_general(xb, W_ref[...], (((1,), (1,)), ((), ())),
                          preferred_element_type=jnp.float32)
    h = jnp.dot(xb, A_ref[...], preferred_element_type=jnp.float32)
    w128 = lax.dot_general(wT_ref[...], Ex_ref[...], (((0,), (0,)), ((), ())),
                           preferred_element_type=jnp.float32)
    hw = (h * w128).astype(jnp.bfloat16)
    lora = jnp.dot(hw, Bf_ref[...], preferred_element_type=jnp.float32)
    o_ref[...] = acc + _SCALING * lora + b_ref[...]


def _fused_main(xf, Wb, b2, Ab, Bb, wT, Ex, N, D, OUT, E, ER, TB):
    grid = N // TB
    return pl.pallas_call(
        _main_body,
        grid=(grid,),
        in_specs=[
            pl.BlockSpec((TB, D), lambda i: (i, 0)),
            pl.BlockSpec((OUT, D), lambda i: (0, 0)),
            pl.BlockSpec((1, OUT), lambda i: (0, 0)),
            pl.BlockSpec((D, ER), lambda i: (0, 0)),
            pl.BlockSpec((ER, OUT), lambda i: (0, 0)),
            pl.BlockSpec((E, TB), lambda i: (0, i)),
            pl.BlockSpec((E, ER), lambda i: (0, 0)),
        ],
        out_specs=pl.BlockSpec((TB, OUT), lambda i: (i, 0)),
        out_shape=jax.ShapeDtypeStruct((N, OUT), jnp.float32),
    )(xf, Wb, b2, Ab, Bb, wT, Ex)


def kernel(x, base_W, base_b, router_W, router_b, lora_A, lora_B):
    B, S, D = x.shape
    OUT = base_W.shape[0]
    E = router_W.shape[0]
    R = lora_A.shape[2]
    ER = E * R
    N = B * S
    TB = 512

    xf = x.reshape(N, D)
    rb2 = router_b.reshape(E, 1)
    b2 = base_b.reshape(1, OUT)
    Wb = base_W.astype(jnp.bfloat16)
    Ab = lora_A.transpose(1, 0, 2).reshape(D, ER).astype(jnp.bfloat16)
    Bb = lora_B.reshape(ER, OUT).astype(jnp.bfloat16)
    Ex = jnp.repeat(jnp.eye(E, dtype=jnp.float32), R, axis=1)

    lgT = _router_logits_T(xf, router_W, rb2, N, D, E, TB)
    wT = _make_sc_route(N, E)(lgT)
    out = _fused_main(xf, Wb, b2, Ab, Bb, wT, Ex, N, D, OUT, E, ER, TB)
    return out.reshape(B, S, OUT)
